# baseline (speedup 1.0000x reference)
import functools

import jax
import jax.numpy as jnp
from jax import lax
from jax.experimental import pallas as pl
from jax.experimental.pallas import tpu as pltpu
from jax.experimental.pallas import tpu_sc as plsc

BATCH = 16384
LOGN = 14
NUM_CORES = 2
NUM_SUBCORES = 16
NUM_WORKERS = NUM_CORES * NUM_SUBCORES
BPW = BATCH // NUM_WORKERS
WIN = 256
WSHIFT = 8
NBUF = 4

_MESH = dict(core_axis_name="c", subcore_axis_name="s")


def _sort_body(k_ref, v_ref, ok_ref, ov_ref):
  rows = BATCH // 128
  k = k_ref[...]
  v = v_ref[...]
  i = ((jax.lax.broadcasted_iota(jnp.int32, (rows, 128), 0) << 7)
       | jax.lax.broadcasted_iota(jnp.int32, (rows, 128), 1))
  for s in range(1, LOGN + 1):
    for t in range(s - 1, -1, -1):
      j = 1 << t
      axis = 0 if j >= 128 else 1
      sh = j >> 7 if j >= 128 else j
      low = (i & j) == 0
      sel_min = (((i >> t) ^ (i >> s)) & 1) == 0
      kp = jnp.where(low, jnp.roll(k, -sh, axis=axis),
                     jnp.roll(k, sh, axis=axis))
      vp = jnp.where(low, jnp.roll(v, -sh, axis=axis),
                     jnp.roll(v, sh, axis=axis))
      a = jnp.where(sel_min, kp, k)
      b = jnp.where(sel_min, k, kp)
      take = a < b
      k = jnp.where(take, kp, k)
      v = jnp.where(take, vp, v)
  ok_ref[...] = k
  ov_ref[...] = v


def _sort_pairs(keys, vals):
  rows = BATCH // 128
  ok, ov = pl.pallas_call(
      _sort_body,
      out_shape=(jax.ShapeDtypeStruct((rows, 128), jnp.int32),
                 jax.ShapeDtypeStruct((rows, 128), jnp.int32)),
  )(keys.reshape(rows, 128), vals.reshape(rows, 128))
  return ok.reshape(BATCH), ov.reshape(BATCH)


def _walk(sorted_u, sorted_i, ut3, it3, uc3, ic3, n_user, n_item):
  mesh = plsc.VectorSubcoreMesh(**_MESH)
  out_types = (
      jax.ShapeDtypeStruct((BATCH * 64,), jnp.float32),
      jax.ShapeDtypeStruct((BATCH * 64,), jnp.float32),
      jax.ShapeDtypeStruct((BATCH * 32,), jnp.float32),
      jax.ShapeDtypeStruct((BATCH * 32,), jnp.float32),
  )
  half = BPW // 2

  @functools.partial(
      pl.kernel,
      mesh=mesh,
      out_type=out_types,
      compiler_params=pltpu.CompilerParams(
          needs_layout_passes=False, disable_bounds_checks=True),
      scratch_types=[
          pltpu.VMEM((BPW + 16,), jnp.int32),
      ] + [pltpu.VMEM((8, 8, WIN), jnp.float32) for _ in range(NBUF)]
        + [pltpu.VMEM((4, 8, WIN), jnp.float32) for _ in range(NBUF)]
        + [
          pltpu.VMEM((half * 64,), jnp.float32),
          pltpu.VMEM((half * 32,), jnp.float32),
      ] + [pltpu.SemaphoreType.DMA for _ in range(2 * NBUF)],
  )
  def k(su_hbm, si_hbm, ut_hbm, it_hbm, uc_hbm, ic_hbm,
        su_out, si_out, uc_out, ic_out,
        idx_s, *bufs):
    wt = bufs[0:NBUF]
    wc = bufs[NBUF:2 * NBUF]
    st64, st32 = bufs[2 * NBUF], bufs[2 * NBUF + 1]
    semt = bufs[2 * NBUF + 2:2 * NBUF + 2 + NBUF]
    semc = bufs[2 * NBUF + 2 + NBUF:]
    wid = lax.axis_index("s") * NUM_CORES + lax.axis_index("c")
    base = pl.multiple_of(wid * BPW, BPW)
    giota = jax.lax.iota(jnp.int32, 16) >> 3
    citer = jax.lax.iota(jnp.int32, 16) & 7

    def phase(sidx_hbm, tab_hbm, cross_hbm, out64, out32, n_rows):
      last_base = ((n_rows - 1) >> WSHIFT) << WSHIFT
      llen = (n_rows - last_base + 127) & ~127
      ob64 = pl.multiple_of(base * 64, BPW * 64)
      ob32 = pl.multiple_of(base * 32, BPW * 32)
      pltpu.sync_copy(sidx_hbm.at[pl.ds(base, BPW)],
                      idx_s.at[pl.ds(0, BPW)])
      idx_s[pl.ds(BPW, 16)] = jnp.broadcast_to(jnp.int32(0x7FFFFFFF), (16,))
      first = idx_s[pl.ds(0, 16)][0]
      last = idx_s[pl.ds(BPW - 16, 16)][15]
      w0 = first >> WSHIFT
      nw = (last >> WSHIFT) - w0 + 1
      nw2 = nw + ((-nw) % NBUF)

      def wbase_of(t):
        return pl.multiple_of(jnp.minimum((w0 + t) << WSHIFT,
                                          jnp.int32(last_base)), 128)

      def issue(t, slot):
        b = wbase_of(t)

        @pl.when(b == last_base)
        def _():
          pltpu.async_copy(tab_hbm.at[:, :, pl.ds(b, llen)],
                           wt[slot].at[:, :, pl.ds(0, llen)], semt[slot])
          pltpu.async_copy(cross_hbm.at[:, :, pl.ds(b, llen)],
                           wc[slot].at[:, :, pl.ds(0, llen)], semc[slot])

        @pl.when(b != last_base)
        def _():
          pltpu.async_copy(tab_hbm.at[:, :, pl.ds(b, WIN)], wt[slot],
                           semt[slot])
          pltpu.async_copy(cross_hbm.at[:, :, pl.ds(b, WIN)], wc[slot],
                           semc[slot])

      def wait(t, slot):
        b = wbase_of(t)

        @pl.when(b == last_base)
        def _():
          pltpu.make_async_copy(tab_hbm.at[:, :, pl.ds(0, llen)],
                                wt[slot].at[:, :, pl.ds(0, llen)],
                                semt[slot]).wait()
          pltpu.make_async_copy(cross_hbm.at[:, :, pl.ds(0, llen)],
                                wc[slot].at[:, :, pl.ds(0, llen)],
                                semc[slot]).wait()

        @pl.when(b != last_base)
        def _():
          pltpu.make_async_copy(tab_hbm.at[:, :, pl.ds(0, WIN)], wt[slot],
                                semt[slot]).wait()
          pltpu.make_async_copy(cross_hbm.at[:, :, pl.ds(0, WIN)], wc[slot],
                                semc[slot]).wait()

      for b0 in range(NBUF):
        if b0 == 0:
          issue(0, 0)
        else:
          @pl.when(nw2 > b0)
          def _(b0=b0):
            issue(b0, b0)

      def guarded_extract(c, wb, slot):
        c16 = pl.multiple_of(c * 16, 16)
        v = idx_s[pl.ds(c16, 16)]
        for kk in range(16):
          ik = v[kk]

          @pl.when((ik >> WSHIFT) << WSHIFT == wb)
          def _(ik=ik, kk=kk):
            r = jnp.broadcast_to(ik - wb, (16,))
            ph = (c16 + kk) & (half - 1)
            o64 = pl.multiple_of(ph * 64, 16)
            o32 = pl.multiple_of(ph * 32, 16)
            for q in range(4):
              st64[pl.ds(o64 + 16 * q, 16)] = (
                  plsc.load_gather(wt[slot], [2 * q + giota, citer, r]))
            for q in range(2):
              st32[pl.ds(o32 + 16 * q, 16)] = (
                  plsc.load_gather(wc[slot], [2 * q + giota, citer, r]))

      def process(t, slot, c):
        wait(t, slot)
        wb = wbase_of(t)
        wend = wb + WIN

        def cond(cc):
          cb = pl.multiple_of(cc * 16, 16)
          return idx_s[pl.ds(cb, 16)][15] < wend

        def body(cc):
          guarded_extract(cc, wb, slot)

          @pl.when(cc == (half // 16) - 1)
          def _():
            pltpu.sync_copy(st64, out64.at[pl.ds(ob64, half * 64)])
            pltpu.sync_copy(st32, out32.at[pl.ds(ob32, half * 32)])

          return cc + 1

        c = lax.while_loop(cond, body, c)
        guarded_extract(c, wb, slot)

        @pl.when(t + NBUF < nw2)
        def _():
          issue(t + NBUF, slot)

        return c

      def outer(h, c):
        for b0 in range(NBUF):
          c = process(NBUF * h + b0, b0, c)
        return c

      lax.fori_loop(0, nw2 // NBUF, outer, jnp.int32(0))
      pltpu.sync_copy(st64, out64.at[pl.ds(ob64 + half * 64, half * 64)])
      pltpu.sync_copy(st32, out32.at[pl.ds(ob32 + half * 32, half * 32)])

    phase(su_hbm, ut_hbm, uc_hbm, su_out, uc_out, n_user)
    phase(si_hbm, it_hbm, ic_hbm, si_out, ic_out, n_item)

  return k(sorted_u, sorted_i, ut3, it3, uc3, ic3)


def _unpermute(ju, ji, su_rows, si_rows, uc_rows, ic_rows):
  mesh = plsc.VectorSubcoreMesh(**_MESH)
  out_types = (
      jax.ShapeDtypeStruct((BATCH, 64), jnp.float32),
      jax.ShapeDtypeStruct((BATCH, 64), jnp.float32),
      jax.ShapeDtypeStruct((BATCH, 32), jnp.float32),
      jax.ShapeDtypeStruct((BATCH, 32), jnp.float32),
  )

  @functools.partial(
      pl.kernel,
      mesh=mesh,
      out_type=out_types,
      compiler_params=pltpu.CompilerParams(
          needs_layout_passes=False, use_tc_tiling_on_sc=False),
      scratch_types=[
          pltpu.VMEM((BPW // 128, 128), jnp.int32),
          pltpu.VMEM((BPW, 64), jnp.float32),
          pltpu.VMEM((BPW, 32), jnp.float32),
          pltpu.SemaphoreType.DMA,
      ],
  )
  def k(ju_hbm, ji_hbm, su_hbm, si_hbm, uc_hbm, ic_hbm,
        ue_out, ie_out, cu_out, ci_out,
        jv, rows64, rows32, sem):
    wid = lax.axis_index("s") * NUM_CORES + lax.axis_index("c")
    base = pl.multiple_of(wid * BPW, BPW)

    def pair(j_hbm, rows_hbm, cross_hbm, out64, out32):
      for c in range(BPW // 128):
        pltpu.sync_copy(j_hbm.at[pl.ds(base + c * 128, 128)], jv.at[c])
      pltpu.sync_copy(rows_hbm.at[pl.ds(base, BPW)], rows64)
      pltpu.sync_copy(cross_hbm.at[pl.ds(base, BPW)], rows32)
      for c in range(BPW // 128):
        csl = pl.ds(c * 128, 128)
        pltpu.async_copy(rows64.at[csl], out64.at[jv.at[c]], sem)
        pltpu.async_copy(rows32.at[csl], out32.at[jv.at[c]], sem)
      pltpu.make_async_copy(rows64, out64.at[pl.ds(0, BPW)], sem).wait()
      pltpu.make_async_copy(rows32, out32.at[pl.ds(0, BPW)], sem).wait()

    pair(ju_hbm, su_hbm, uc_hbm, ue_out, cu_out)
    pair(ji_hbm, si_hbm, ic_hbm, ie_out, ci_out)

  return k(ju, ji, su_rows, si_rows, uc_rows, ic_rows)


def _cross_body(cu_ref, ci_ref, o_ref):
  o_ref[...] = jnp.sum(cu_ref[...] * ci_ref[...], axis=1, keepdims=True)


def _cross_tc(cu, ci):
  return pl.pallas_call(
      _cross_body,
      out_shape=jax.ShapeDtypeStruct((BATCH, 1), jnp.float32),
  )(cu, ci)


def kernel(users, items, user_table, item_table, user_cross_table,
           item_cross_table):
  n_user = user_table.shape[0]
  n_item = item_table.shape[0]
  pos = jnp.arange(BATCH, dtype=jnp.int32)
  su, ju = _sort_pairs(users, pos)
  si, ji = _sort_pairs(items, pos)
  ut3 = user_table.T.reshape(8, 8, n_user)
  it3 = item_table.T.reshape(8, 8, n_item)
  uc3 = user_cross_table.T.reshape(4, 8, n_user)
  ic3 = item_cross_table.T.reshape(4, 8, n_item)
  sur, sir, ucr, icr = _walk(su, si, ut3, it3, uc3, ic3, n_user, n_item)
  ue, ie, cu, ci = _unpermute(ju, ji,
                              sur.reshape(BATCH, 64), sir.reshape(BATCH, 64),
                              ucr.reshape(BATCH, 32), icr.reshape(BATCH, 32))
  cross = _cross_tc(cu, ci)
  return (ue, ie, cu, ci, cross)


# speedup vs baseline: 1.1446x; 1.1446x over previous
import functools

import jax
import jax.numpy as jnp
from jax import lax
from jax.experimental import pallas as pl
from jax.experimental.pallas import tpu as pltpu
from jax.experimental.pallas import tpu_sc as plsc

BATCH = 16384
LOGN = 14
NUM_CORES = 2
NUM_SUBCORES = 16
NUM_WORKERS = NUM_CORES * NUM_SUBCORES
BPW = BATCH // NUM_WORKERS
WIN = 512
WSHIFT = 9

_MESH = dict(core_axis_name="c", subcore_axis_name="s")


def _sort_body(k_ref, v_ref, ok_ref, ov_ref):
  rows = BATCH // 128
  k = k_ref[...]
  v = v_ref[...]
  i = ((jax.lax.broadcasted_iota(jnp.int32, (rows, 128), 0) << 7)
       | jax.lax.broadcasted_iota(jnp.int32, (rows, 128), 1))
  for s in range(1, LOGN + 1):
    for t in range(s - 1, -1, -1):
      j = 1 << t
      axis = 0 if j >= 128 else 1
      sh = j >> 7 if j >= 128 else j
      low = (i & j) == 0
      sel_min = (((i >> t) ^ (i >> s)) & 1) == 0
      kp = jnp.where(low, jnp.roll(k, -sh, axis=axis),
                     jnp.roll(k, sh, axis=axis))
      vp = jnp.where(low, jnp.roll(v, -sh, axis=axis),
                     jnp.roll(v, sh, axis=axis))
      a = jnp.where(sel_min, kp, k)
      b = jnp.where(sel_min, k, kp)
      take = a < b
      k = jnp.where(take, kp, k)
      v = jnp.where(take, vp, v)
  ok_ref[...] = k
  ov_ref[...] = v


def _sort_pairs(keys, vals):
  rows = BATCH // 128
  ok, ov = pl.pallas_call(
      _sort_body,
      out_shape=(jax.ShapeDtypeStruct((rows, 128), jnp.int32),
                 jax.ShapeDtypeStruct((rows, 128), jnp.int32)),
  )(keys.reshape(rows, 128), vals.reshape(rows, 128))
  return ok.reshape(BATCH), ov.reshape(BATCH)


def _walk(sorted_u, sorted_i, ut3, it3, uc3, ic3, n_user, n_item):
  mesh = plsc.VectorSubcoreMesh(**_MESH)
  out_types = (
      jax.ShapeDtypeStruct((BATCH * 64,), jnp.float32),
      jax.ShapeDtypeStruct((BATCH * 64,), jnp.float32),
      jax.ShapeDtypeStruct((BATCH * 32,), jnp.float32),
      jax.ShapeDtypeStruct((BATCH * 32,), jnp.float32),
  )
  half = BPW // 2

  @functools.partial(
      pl.kernel,
      mesh=mesh,
      out_type=out_types,
      compiler_params=pltpu.CompilerParams(
          needs_layout_passes=False, disable_bounds_checks=True),
      scratch_types=[
          pltpu.VMEM((BPW + 16,), jnp.int32),
          pltpu.VMEM((8, 8, WIN), jnp.float32),
          pltpu.VMEM((8, 8, WIN), jnp.float32),
          pltpu.VMEM((4, 8, WIN), jnp.float32),
          pltpu.VMEM((4, 8, WIN), jnp.float32),
          pltpu.VMEM((half * 64,), jnp.float32),
          pltpu.VMEM((half * 32,), jnp.float32),
          pltpu.SemaphoreType.DMA,
          pltpu.SemaphoreType.DMA,
          pltpu.SemaphoreType.DMA,
          pltpu.SemaphoreType.DMA,
      ],
  )
  def k(su_hbm, si_hbm, ut_hbm, it_hbm, uc_hbm, ic_hbm,
        su_out, si_out, uc_out, ic_out,
        idx_s, wt0, wt1, wc0, wc1, st64, st32,
        semt0, semt1, semc0, semc1):
    wid = lax.axis_index("s") * NUM_CORES + lax.axis_index("c")
    base = pl.multiple_of(wid * BPW, BPW)
    wt = (wt0, wt1)
    wc = (wc0, wc1)
    semt = (semt0, semt1)
    semc = (semc0, semc1)
    giota = jax.lax.iota(jnp.int32, 16) >> 3
    citer = jax.lax.iota(jnp.int32, 16) & 7

    def phase(sidx_hbm, tab_hbm, cross_hbm, out64, out32, n_rows):
      last_base = ((n_rows - 1) >> WSHIFT) << WSHIFT
      llen = (n_rows - last_base + 127) & ~127
      ob64 = pl.multiple_of(base * 64, BPW * 64)
      ob32 = pl.multiple_of(base * 32, BPW * 32)
      pltpu.sync_copy(sidx_hbm.at[pl.ds(base, BPW)],
                      idx_s.at[pl.ds(0, BPW)])
      idx_s[pl.ds(BPW, 16)] = jnp.broadcast_to(jnp.int32(0x7FFFFFFF), (16,))
      first = idx_s[pl.ds(0, 16)][0]
      last = idx_s[pl.ds(BPW - 16, 16)][15]
      w0 = first >> WSHIFT
      nw = (last >> WSHIFT) - w0 + 1
      nw2 = nw + (nw & 1)

      def wbase_of(t):
        return pl.multiple_of(jnp.minimum((w0 + t) << WSHIFT,
                                          jnp.int32(last_base)), 128)

      def issue(t, slot):
        b = wbase_of(t)

        @pl.when(b == last_base)
        def _():
          pltpu.async_copy(tab_hbm.at[:, :, pl.ds(b, llen)],
                           wt[slot].at[:, :, pl.ds(0, llen)], semt[slot])
          pltpu.async_copy(cross_hbm.at[:, :, pl.ds(b, llen)],
                           wc[slot].at[:, :, pl.ds(0, llen)], semc[slot])

        @pl.when(b != last_base)
        def _():
          pltpu.async_copy(tab_hbm.at[:, :, pl.ds(b, WIN)], wt[slot],
                           semt[slot])
          pltpu.async_copy(cross_hbm.at[:, :, pl.ds(b, WIN)], wc[slot],
                           semc[slot])

      def wait(t, slot):
        b = wbase_of(t)

        @pl.when(b == last_base)
        def _():
          pltpu.make_async_copy(tab_hbm.at[:, :, pl.ds(0, llen)],
                                wt[slot].at[:, :, pl.ds(0, llen)],
                                semt[slot]).wait()
          pltpu.make_async_copy(cross_hbm.at[:, :, pl.ds(0, llen)],
                                wc[slot].at[:, :, pl.ds(0, llen)],
                                semc[slot]).wait()

        @pl.when(b != last_base)
        def _():
          pltpu.make_async_copy(tab_hbm.at[:, :, pl.ds(0, WIN)], wt[slot],
                                semt[slot]).wait()
          pltpu.make_async_copy(cross_hbm.at[:, :, pl.ds(0, WIN)], wc[slot],
                                semc[slot]).wait()

      issue(0, 0)

      @pl.when(nw2 > 1)
      def _():
        issue(1, 1)

      def guarded_extract(c, wb, slot):
        c16 = pl.multiple_of(c * 16, 16)
        v = idx_s[pl.ds(c16, 16)]
        for kk in range(16):
          ik = v[kk]

          @pl.when((ik >> WSHIFT) << WSHIFT == wb)
          def _(ik=ik, kk=kk):
            r = jnp.broadcast_to(ik - wb, (16,))
            ph = (c16 + kk) & (half - 1)
            o64 = pl.multiple_of(ph * 64, 16)
            o32 = pl.multiple_of(ph * 32, 16)
            for q in range(4):
              st64[pl.ds(o64 + 16 * q, 16)] = (
                  plsc.load_gather(wt[slot], [2 * q + giota, citer, r]))
            for q in range(2):
              st32[pl.ds(o32 + 16 * q, 16)] = (
                  plsc.load_gather(wc[slot], [2 * q + giota, citer, r]))

      def process(t, slot, c):
        wait(t, slot)
        wb = wbase_of(t)
        wend = wb + WIN

        def cond(cc):
          cb = pl.multiple_of(cc * 16, 16)
          return idx_s[pl.ds(cb, 16)][15] < wend

        def body(cc):
          guarded_extract(cc, wb, slot)

          @pl.when(cc == (half // 16) - 1)
          def _():
            pltpu.sync_copy(st64, out64.at[pl.ds(ob64, half * 64)])
            pltpu.sync_copy(st32, out32.at[pl.ds(ob32, half * 32)])

          return cc + 1

        c = lax.while_loop(cond, body, c)
        guarded_extract(c, wb, slot)

        @pl.when(t + 2 < nw2)
        def _():
          issue(t + 2, slot)

        return c

      def outer(h, c):
        c = process(2 * h, 0, c)
        c = process(2 * h + 1, 1, c)
        return c

      lax.fori_loop(0, nw2 >> 1, outer, jnp.int32(0))
      pltpu.sync_copy(st64, out64.at[pl.ds(ob64 + half * 64, half * 64)])
      pltpu.sync_copy(st32, out32.at[pl.ds(ob32 + half * 32, half * 32)])

    phase(su_hbm, ut_hbm, uc_hbm, su_out, uc_out, n_user)
    phase(si_hbm, it_hbm, ic_hbm, si_out, ic_out, n_item)

  return k(sorted_u, sorted_i, ut3, it3, uc3, ic3)


def _unpermute(ju, ji, su_rows, si_rows, uc_rows, ic_rows):
  mesh = plsc.VectorSubcoreMesh(**_MESH)
  out_types = (
      jax.ShapeDtypeStruct((BATCH, 64), jnp.float32),
      jax.ShapeDtypeStruct((BATCH, 64), jnp.float32),
      jax.ShapeDtypeStruct((BATCH, 32), jnp.float32),
      jax.ShapeDtypeStruct((BATCH, 32), jnp.float32),
  )

  @functools.partial(
      pl.kernel,
      mesh=mesh,
      out_type=out_types,
      compiler_params=pltpu.CompilerParams(
          needs_layout_passes=False, use_tc_tiling_on_sc=False),
      scratch_types=[
          pltpu.VMEM((BPW // 128, 128), jnp.int32),
          pltpu.VMEM((BPW, 64), jnp.float32),
          pltpu.VMEM((BPW, 32), jnp.float32),
          pltpu.SemaphoreType.DMA,
      ],
  )
  def k(ju_hbm, ji_hbm, su_hbm, si_hbm, uc_hbm, ic_hbm,
        ue_out, ie_out, cu_out, ci_out,
        jv, rows64, rows32, sem):
    wid = lax.axis_index("s") * NUM_CORES + lax.axis_index("c")
    base = pl.multiple_of(wid * BPW, BPW)

    def pair(j_hbm, rows_hbm, cross_hbm, out64, out32):
      for c in range(BPW // 128):
        pltpu.sync_copy(j_hbm.at[pl.ds(base + c * 128, 128)], jv.at[c])
      pltpu.sync_copy(rows_hbm.at[pl.ds(base, BPW)], rows64)
      pltpu.sync_copy(cross_hbm.at[pl.ds(base, BPW)], rows32)
      for c in range(BPW // 128):
        csl = pl.ds(c * 128, 128)
        pltpu.async_copy(rows64.at[csl], out64.at[jv.at[c]], sem)
        pltpu.async_copy(rows32.at[csl], out32.at[jv.at[c]], sem)
      pltpu.make_async_copy(rows64, out64.at[pl.ds(0, BPW)], sem).wait()
      pltpu.make_async_copy(rows32, out32.at[pl.ds(0, BPW)], sem).wait()

    pair(ju_hbm, su_hbm, uc_hbm, ue_out, cu_out)
    pair(ji_hbm, si_hbm, ic_hbm, ie_out, ci_out)

  return k(ju, ji, su_rows, si_rows, uc_rows, ic_rows)


def _cross_body(cu_ref, ci_ref, o_ref):
  o_ref[...] = jnp.sum(cu_ref[...] * ci_ref[...], axis=1, keepdims=True)


def _cross_tc(cu, ci):
  return pl.pallas_call(
      _cross_body,
      out_shape=jax.ShapeDtypeStruct((BATCH, 1), jnp.float32),
  )(cu, ci)


def kernel(users, items, user_table, item_table, user_cross_table,
           item_cross_table):
  n_user = user_table.shape[0]
  n_item = item_table.shape[0]
  pos = jnp.arange(BATCH, dtype=jnp.int32)
  su, ju = _sort_pairs(users, pos)
  si, ji = _sort_pairs(items, pos)
  ut3 = user_table.T.reshape(8, 8, n_user)
  it3 = item_table.T.reshape(8, 8, n_item)
  uc3 = user_cross_table.T.reshape(4, 8, n_user)
  ic3 = item_cross_table.T.reshape(4, 8, n_item)
  sur, sir, ucr, icr = _walk(su, si, ut3, it3, uc3, ic3, n_user, n_item)
  ue, ie, cu, ci = _unpermute(ju, ji,
                              sur.reshape(BATCH, 64), sir.reshape(BATCH, 64),
                              ucr.reshape(BATCH, 32), icr.reshape(BATCH, 32))
  cross = _cross_tc(cu, ci)
  return (ue, ie, cu, ci, cross)
